# Initial kernel scaffold; baseline (speedup 1.0000x reference)
import jax
import jax.numpy as jnp
from jax.experimental import pallas as pl


def kernel(indices, codebook):
    raise NotImplementedError("write your pallas kernel here")


# speedup vs baseline: 4.0386x; 4.0386x over previous
import functools

import jax
import jax.numpy as jnp
from jax import lax
from jax.experimental import pallas as pl
from jax.experimental.pallas import tpu as pltpu
from jax.experimental.pallas import tpu_sc as plsc


def _make_gather(B, K, D, NW, C):
    b_per_w = B // NW
    n_chunks = b_per_w // C
    mesh = plsc.VectorSubcoreMesh(core_axis_name="c", subcore_axis_name="s")

    @functools.partial(
        pl.kernel,
        mesh=mesh,
        compiler_params=pltpu.CompilerParams(use_tc_tiling_on_sc=False),
        out_type=jax.ShapeDtypeStruct((B, D), jnp.float32),
        scratch_types=[
            pltpu.VMEM((C,), jnp.int32),
            pltpu.VMEM((C, D), jnp.float32),
            pltpu.SemaphoreType.DMA,
        ],
    )
    def gather_kernel(idx_hbm, table_hbm, out_hbm, idx_v, rows_v, sem):
        wid = lax.axis_index("s") * 2 + lax.axis_index("c")
        base = wid * b_per_w

        def chunk_body(i, carry):
            off = base + i * C
            pltpu.sync_copy(idx_hbm.at[pl.ds(off, C)], idx_v)
            pltpu.async_copy(table_hbm.at[idx_v], rows_v, sem).wait()
            pltpu.sync_copy(rows_v, out_hbm.at[pl.ds(off, C)])
            return carry

        lax.fori_loop(0, n_chunks, chunk_body, 0)

    return gather_kernel


def kernel(indices, codebook):
    B0, T = indices.shape
    K, D = codebook.shape
    B = B0 * T
    NW = 32
    C = 512
    flat_idx = indices.reshape(B)
    out = _make_gather(B, K, D, NW, C)(flat_idx, codebook)
    return out.reshape(B0, T, D)


# speedup vs baseline: 4.2658x; 1.0563x over previous
import functools

import jax
import jax.numpy as jnp
from jax import lax
from jax.experimental import pallas as pl
from jax.experimental.pallas import tpu as pltpu
from jax.experimental.pallas import tpu_sc as plsc

_NW = 32


def _make_gather(B, K, D, C, NBUF):
    b_per_w = B // _NW
    n_chunks = b_per_w // C
    mesh = plsc.VectorSubcoreMesh(core_axis_name="c", subcore_axis_name="s")

    @functools.partial(
        pl.kernel,
        mesh=mesh,
        compiler_params=pltpu.CompilerParams(use_tc_tiling_on_sc=False),
        out_type=jax.ShapeDtypeStruct((B, D), jnp.float32),
        scratch_types=[
            pltpu.VMEM((b_per_w,), jnp.int32),
            pltpu.VMEM((NBUF, C, D), jnp.float32),
            [pltpu.SemaphoreType.DMA] * NBUF,
            [pltpu.SemaphoreType.DMA] * NBUF,
        ],
    )
    def gather_kernel(idx_hbm, table_hbm, out_hbm, idx_v, rows_v, gsems, wsems):
        wid = lax.axis_index("s") * 2 + lax.axis_index("c")
        base = wid * b_per_w
        pltpu.sync_copy(idx_hbm.at[pl.ds(base, b_per_w)], idx_v)

        gd = [None] * n_chunks
        wd = [None] * n_chunks
        for c in range(n_chunks):
            b = c % NBUF
            if c >= NBUF:
                wd[c - NBUF].wait()
            gd[c] = pltpu.async_copy(
                table_hbm.at[idx_v.at[pl.ds(c * C, C)]], rows_v.at[b], gsems[b]
            )
            if c >= 1:
                p = c - 1
                gd[p].wait()
                wd[p] = pltpu.async_copy(
                    rows_v.at[p % NBUF],
                    out_hbm.at[pl.ds(base + p * C, C)],
                    wsems[p % NBUF],
                )
        last = n_chunks - 1
        gd[last].wait()
        wd[last] = pltpu.async_copy(
            rows_v.at[last % NBUF],
            out_hbm.at[pl.ds(base + last * C, C)],
            wsems[last % NBUF],
        )
        for c in range(n_chunks - NBUF, n_chunks):
            wd[c].wait()

    return gather_kernel


def kernel(indices, codebook):
    B0, T = indices.shape
    K, D = codebook.shape
    B = B0 * T
    flat_idx = indices.reshape(B)
    out = _make_gather(B, K, D, C=512, NBUF=3)(flat_idx, codebook)
    return out.reshape(B0, T, D)


# speedup vs baseline: 4.2781x; 1.0029x over previous
import functools

import jax
import jax.numpy as jnp
from jax import lax
from jax.experimental import pallas as pl
from jax.experimental.pallas import tpu as pltpu
from jax.experimental.pallas import tpu_sc as plsc

_NW = 32


def _make_gather(B0, T, K, D, C, NBUF):
    B = B0 * T
    b_per_w = B // _NW
    n_chunks = b_per_w // C
    chunks_per_row = T // C
    mesh = plsc.VectorSubcoreMesh(core_axis_name="c", subcore_axis_name="s")

    @functools.partial(
        pl.kernel,
        mesh=mesh,
        compiler_params=pltpu.CompilerParams(use_tc_tiling_on_sc=False),
        out_type=jax.ShapeDtypeStruct((B0, T, D), jnp.float32),
        scratch_types=[
            pltpu.VMEM((b_per_w,), jnp.int32),
            pltpu.VMEM((NBUF, C, D), jnp.float32),
            [pltpu.SemaphoreType.DMA] * NBUF,
            [pltpu.SemaphoreType.DMA] * NBUF,
        ],
    )
    def gather_kernel(idx_hbm, table_hbm, out_hbm, idx_v, rows_v, gsems, wsems):
        wid = lax.axis_index("s") * 2 + lax.axis_index("c")
        base = wid * b_per_w
        pltpu.sync_copy(idx_hbm.at[pl.ds(base, b_per_w)], idx_v)

        def out_ref(c):
            flat = base + c * C
            img = flat // T
            tok = flat % T
            return out_hbm.at[img, pl.ds(tok, C)]

        gd = [None] * n_chunks
        wd = [None] * n_chunks
        for c in range(n_chunks):
            b = c % NBUF
            if c >= NBUF:
                wd[c - NBUF].wait()
            gd[c] = pltpu.async_copy(
                table_hbm.at[idx_v.at[pl.ds(c * C, C)]], rows_v.at[b], gsems[b]
            )
            if c >= 1:
                p = c - 1
                gd[p].wait()
                wd[p] = pltpu.async_copy(
                    rows_v.at[p % NBUF], out_ref(p), wsems[p % NBUF]
                )
        last = n_chunks - 1
        gd[last].wait()
        wd[last] = pltpu.async_copy(
            rows_v.at[last % NBUF], out_ref(last), wsems[last % NBUF]
        )
        for c in range(n_chunks - NBUF, n_chunks):
            wd[c].wait()

    return gather_kernel


def kernel(indices, codebook):
    B0, T = indices.shape
    K, D = codebook.shape
    flat_idx = indices.reshape(B0 * T)
    return _make_gather(B0, T, K, D, C=512, NBUF=3)(flat_idx, codebook)


# speedup vs baseline: 4.6899x; 1.0963x over previous
import functools

import jax
import jax.numpy as jnp
from jax import lax
from jax.experimental import pallas as pl
from jax.experimental.pallas import tpu as pltpu
from jax.experimental.pallas import tpu_sc as plsc

_NW = 32


def _make_gather(B0, T, K, D):
    FS = 8
    NSLAB = D // FS
    NQ = _NW // NSLAB
    TOK = T // NQ
    IMG_BLK = 8
    NBLK = B0 // IMG_BLK
    mesh = plsc.VectorSubcoreMesh(core_axis_name="c", subcore_axis_name="s")

    @functools.partial(
        pl.kernel,
        mesh=mesh,
        compiler_params=pltpu.CompilerParams(needs_layout_passes=False),
        out_type=jax.ShapeDtypeStruct((B0, D, T), jnp.float32),
        scratch_types=[
            pltpu.VMEM((FS, K), jnp.float32),
            pltpu.VMEM((IMG_BLK, TOK), jnp.int32),
            pltpu.VMEM((IMG_BLK, TOK), jnp.int32),
            pltpu.VMEM((IMG_BLK, FS, TOK), jnp.float32),
            pltpu.VMEM((IMG_BLK, FS, TOK), jnp.float32),
            [pltpu.SemaphoreType.DMA] * 2,
            [pltpu.SemaphoreType.DMA] * 2,
        ],
    )
    def gather_kernel(
        idx_hbm, ct_hbm, out_hbm, trows, idx0, idx1, ob0, ob1, isems, osems
    ):
        wid = lax.axis_index("s") * 2 + lax.axis_index("c")
        slab = wid // NQ
        f0 = slab * FS
        t0 = (wid % NQ) * TOK
        pltpu.sync_copy(ct_hbm.at[pl.ds(f0, FS)], trows)

        idx_bufs = (idx0, idx1)
        obufs = (ob0, ob1)
        ic = [None, None]
        oc = [None, None]
        ic[0] = pltpu.async_copy(
            idx_hbm.at[pl.ds(0, IMG_BLK), pl.ds(t0, TOK)], idx_bufs[0], isems[0]
        )
        for blk in range(NBLK):
            buf = blk % 2
            idx_v = idx_bufs[buf]
            obuf = obufs[buf]
            if blk + 1 < NBLK:
                ic[1 - buf] = pltpu.async_copy(
                    idx_hbm.at[pl.ds((blk + 1) * IMG_BLK, IMG_BLK), pl.ds(t0, TOK)],
                    idx_bufs[1 - buf],
                    isems[1 - buf],
                )
            ic[buf].wait()
            if blk >= 2:
                oc[buf].wait()

            @pl.loop(0, IMG_BLK)
            def _img(il, _idx_v=idx_v, _obuf=obuf):
                @pl.loop(0, TOK // 16, unroll=4)
                def _grp(g, _il=il, _idx_v=_idx_v, _obuf=_obuf):
                    iv = _idx_v[_il, pl.ds(g * 16, 16)]
                    zero = jnp.zeros_like(iv)
                    for fl in range(FS):
                        _obuf[_il, fl, pl.ds(g * 16, 16)] = plsc.load_gather(
                            trows, [zero + fl, iv]
                        )

            oc[buf] = pltpu.async_copy(
                obuf,
                out_hbm.at[
                    pl.ds(blk * IMG_BLK, IMG_BLK), pl.ds(f0, FS), pl.ds(t0, TOK)
                ],
                osems[buf],
            )
        oc[0].wait()
        oc[1].wait()

    return gather_kernel


def kernel(indices, codebook):
    B0, T = indices.shape
    K, D = codebook.shape
    ct = codebook.T
    out = _make_gather(B0, T, K, D)(indices, ct)
    return out.transpose(0, 2, 1)


# speedup vs baseline: 12.0098x; 2.5608x over previous
import functools

import jax
import jax.numpy as jnp
from jax import lax
from jax.experimental import pallas as pl
from jax.experimental.pallas import tpu as pltpu
from jax.experimental.pallas import tpu_sc as plsc

_NW = 32


def _make_gather(B0, T, K, D):
    FS = 8
    NSLAB = D // FS
    NQ = _NW // NSLAB
    TOK = T // NQ
    IMG_BLK = 8
    NBLK = B0 // IMG_BLK
    mesh = plsc.VectorSubcoreMesh(core_axis_name="c", subcore_axis_name="s")

    @functools.partial(
        pl.kernel,
        mesh=mesh,
        compiler_params=pltpu.CompilerParams(needs_layout_passes=False),
        out_type=jax.ShapeDtypeStruct((B0, D, T), jnp.float32),
        scratch_types=[
            pltpu.VMEM((FS, K), jnp.float32),
            pltpu.VMEM((IMG_BLK, TOK), jnp.int32),
            pltpu.VMEM((IMG_BLK, TOK), jnp.int32),
            pltpu.VMEM((IMG_BLK, FS, TOK), jnp.float32),
            pltpu.VMEM((IMG_BLK, FS, TOK), jnp.float32),
            [pltpu.SemaphoreType.DMA] * 2,
            [pltpu.SemaphoreType.DMA] * 2,
        ],
    )
    def gather_kernel(
        idx_hbm, ct_hbm, out_hbm, trows, idx0, idx1, ob0, ob1, isems, osems
    ):
        wid = lax.axis_index("s") * 2 + lax.axis_index("c")
        slab = wid // NQ
        f0 = slab * FS
        t0 = (wid % NQ) * TOK
        pltpu.sync_copy(ct_hbm.at[pl.ds(f0, FS)], trows)

        idx_bufs = (idx0, idx1)
        obufs = (ob0, ob1)
        ic = [None, None]
        oc = [None, None]
        ic[0] = pltpu.async_copy(
            idx_hbm.at[pl.ds(0, IMG_BLK), pl.ds(t0, TOK)], idx_bufs[0], isems[0]
        )
        for blk in range(NBLK):
            buf = blk % 2
            idx_v = idx_bufs[buf]
            obuf = obufs[buf]
            if blk + 1 < NBLK:
                ic[1 - buf] = pltpu.async_copy(
                    idx_hbm.at[pl.ds((blk + 1) * IMG_BLK, IMG_BLK), pl.ds(t0, TOK)],
                    idx_bufs[1 - buf],
                    isems[1 - buf],
                )
            ic[buf].wait()
            if blk >= 2:
                oc[buf].wait()

            @pl.loop(0, IMG_BLK)
            def _img(il, _idx_v=idx_v, _obuf=obuf):
                @plsc.parallel_loop(0, TOK // 16, unroll=8)
                def _grp(g, _il=il, _idx_v=_idx_v, _obuf=_obuf):
                    iv = _idx_v[_il, pl.ds(g * 16, 16)]
                    zero = jnp.zeros_like(iv)
                    for fl in range(FS):
                        _obuf[_il, fl, pl.ds(g * 16, 16)] = plsc.load_gather(
                            trows, [zero + fl, iv]
                        )

            oc[buf] = pltpu.async_copy(
                obuf,
                out_hbm.at[
                    pl.ds(blk * IMG_BLK, IMG_BLK), pl.ds(f0, FS), pl.ds(t0, TOK)
                ],
                osems[buf],
            )
        oc[0].wait()
        oc[1].wait()

    return gather_kernel


def kernel(indices, codebook):
    B0, T = indices.shape
    K, D = codebook.shape
    ct = codebook.T
    out = _make_gather(B0, T, K, D)(indices, ct)
    return out.transpose(0, 2, 1)


# speedup vs baseline: 13.2126x; 1.1002x over previous
import functools

import jax
import jax.numpy as jnp
from jax import lax
from jax.experimental import pallas as pl
from jax.experimental.pallas import tpu as pltpu
from jax.experimental.pallas import tpu_sc as plsc

_NW = 32


def _make_gather(B0, T, K, D):
    FS = 8
    NSLAB = D // FS
    NQ = _NW // NSLAB
    TOK = T // NQ
    IMG_BLK = 8
    NBLK = B0 // IMG_BLK
    mesh = plsc.VectorSubcoreMesh(core_axis_name="c", subcore_axis_name="s")

    @functools.partial(
        pl.kernel,
        mesh=mesh,
        compiler_params=pltpu.CompilerParams(needs_layout_passes=False),
        out_type=jax.ShapeDtypeStruct((B0, D, T), jnp.float32),
        scratch_types=[
            pltpu.VMEM((FS, K), jnp.float32),
            pltpu.VMEM((IMG_BLK, TOK), jnp.int32),
            pltpu.VMEM((IMG_BLK, TOK), jnp.int32),
            pltpu.VMEM((IMG_BLK, FS, TOK), jnp.float32),
            pltpu.VMEM((IMG_BLK, FS, TOK), jnp.float32),
            [pltpu.SemaphoreType.DMA] * 2,
            [pltpu.SemaphoreType.DMA] * 2,
        ],
    )
    def gather_kernel(
        idx_hbm, ct_hbm, out_hbm, trows, idx0, idx1, ob0, ob1, isems, osems
    ):
        wid = lax.axis_index("s") * 2 + lax.axis_index("c")
        slab = wid // NQ
        f0 = slab * FS
        t0 = (wid % NQ) * TOK
        pltpu.sync_copy(ct_hbm.at[pl.ds(f0, FS)], trows)

        idx_bufs = (idx0, idx1)
        obufs = (ob0, ob1)
        ic = [None, None]
        oc = [None, None]
        ic[0] = pltpu.async_copy(
            idx_hbm.at[pl.ds(0, IMG_BLK), pl.ds(t0, TOK)], idx_bufs[0], isems[0]
        )
        for blk in range(NBLK):
            buf = blk % 2
            idx_v = idx_bufs[buf]
            obuf = obufs[buf]
            if blk + 1 < NBLK:
                ic[1 - buf] = pltpu.async_copy(
                    idx_hbm.at[pl.ds((blk + 1) * IMG_BLK, IMG_BLK), pl.ds(t0, TOK)],
                    idx_bufs[1 - buf],
                    isems[1 - buf],
                )
            ic[buf].wait()
            if blk >= 2:
                oc[buf].wait()

            @plsc.parallel_loop(0, IMG_BLK * (TOK // 16), unroll=8)
            def _grp(i, _idx_v=idx_v, _obuf=obuf):
                il = i >> 4
                g = i & 15
                iv = _idx_v[il, pl.ds(g * 16, 16)]
                zero = jnp.zeros_like(iv)
                for fl in range(FS):
                    _obuf[il, fl, pl.ds(g * 16, 16)] = plsc.load_gather(
                        trows, [zero + fl, iv]
                    )

            oc[buf] = pltpu.async_copy(
                obuf,
                out_hbm.at[
                    pl.ds(blk * IMG_BLK, IMG_BLK), pl.ds(f0, FS), pl.ds(t0, TOK)
                ],
                osems[buf],
            )
        oc[0].wait()
        oc[1].wait()

    return gather_kernel


def kernel(indices, codebook):
    B0, T = indices.shape
    K, D = codebook.shape
    ct = codebook.T
    out = _make_gather(B0, T, K, D)(indices, ct)
    return out.transpose(0, 2, 1)


# speedup vs baseline: 13.4710x; 1.0196x over previous
import functools

import jax
import jax.numpy as jnp
from jax import lax
from jax.experimental import pallas as pl
from jax.experimental.pallas import tpu as pltpu
from jax.experimental.pallas import tpu_sc as plsc

_NW = 32


def _make_gather(B0, T, K, D, unroll=16):
    FS = 8
    NSLAB = D // FS
    NQ = _NW // NSLAB
    TOK = T // NQ
    IMG_BLK = 8
    NBLK = B0 // IMG_BLK
    NGRP = IMG_BLK * (TOK // 16)
    mesh = plsc.VectorSubcoreMesh(core_axis_name="c", subcore_axis_name="s")

    @functools.partial(
        pl.kernel,
        mesh=mesh,
        compiler_params=pltpu.CompilerParams(needs_layout_passes=False),
        out_type=jax.ShapeDtypeStruct((B0, D, T), jnp.float32),
        scratch_types=[
            pltpu.VMEM((FS, K), jnp.float32),
            pltpu.VMEM((IMG_BLK, TOK), jnp.int32),
            pltpu.VMEM((IMG_BLK, TOK), jnp.int32),
            pltpu.VMEM((IMG_BLK, FS, TOK), jnp.float32),
            pltpu.VMEM((IMG_BLK, FS, TOK), jnp.float32),
            [pltpu.SemaphoreType.DMA] * 2,
            [pltpu.SemaphoreType.DMA] * 2,
        ],
    )
    def gather_kernel(
        idx_hbm, ct_hbm, out_hbm, trows, idx0, idx1, ob0, ob1, isems, osems
    ):
        wid = lax.axis_index("s") * 2 + lax.axis_index("c")
        slab = wid // NQ
        f0 = slab * FS
        t0 = (wid % NQ) * TOK
        pltpu.sync_copy(ct_hbm.at[pl.ds(f0, FS)], trows)

        idx_bufs = (idx0, idx1)
        obufs = (ob0, ob1)

        def idx_src(img0):
            return idx_hbm.at[pl.ds(img0, IMG_BLK), pl.ds(t0, TOK)]

        def out_dst(img0):
            return out_hbm.at[
                pl.ds(img0, IMG_BLK), pl.ds(f0, FS), pl.ds(t0, TOK)
            ]

        def compute(idx_v, obuf):
            @plsc.parallel_loop(0, NGRP, unroll=unroll)
            def _grp(i, _idx_v=idx_v, _obuf=obuf):
                il = i >> 4
                g = i & 15
                iv = _idx_v[il, pl.ds(g * 16, 16)]
                zero = jnp.zeros_like(iv)
                for fl in range(FS):
                    _obuf[il, fl, pl.ds(g * 16, 16)] = plsc.load_gather(
                        trows, [zero + fl, iv]
                    )

        def step(blk, b, wait_out):
            nxt = jnp.minimum((blk + 1) * IMG_BLK, B0 - IMG_BLK)
            pltpu.async_copy(idx_src(nxt), idx_bufs[1 - b], isems[1 - b])
            pltpu.make_async_copy(idx_src(0), idx_bufs[b], isems[b]).wait()
            if wait_out:
                pltpu.make_async_copy(obufs[b], out_dst(0), osems[b]).wait()
            compute(idx_bufs[b], obufs[b])
            pltpu.async_copy(obufs[b], out_dst(blk * IMG_BLK), osems[b])

        pltpu.async_copy(idx_src(0), idx_bufs[0], isems[0])
        step(0, 0, False)
        step(1, 1, False)

        @pl.loop(1, NBLK // 2)
        def _pair(p):
            blk = p * 2
            step(blk, 0, True)
            step(blk + 1, 1, True)

        pltpu.make_async_copy(idx_src(0), idx_bufs[0], isems[0]).wait()
        pltpu.make_async_copy(obufs[0], out_dst(0), osems[0]).wait()
        pltpu.make_async_copy(obufs[1], out_dst(0), osems[1]).wait()

    return gather_kernel


def kernel(indices, codebook):
    B0, T = indices.shape
    K, D = codebook.shape
    ct = codebook.T
    out = _make_gather(B0, T, K, D)(indices, ct)
    return out.transpose(0, 2, 1)


# speedup vs baseline: 14.2665x; 1.0590x over previous
import functools

import jax
import jax.numpy as jnp
from jax import lax
from jax.experimental import pallas as pl
from jax.experimental.pallas import tpu as pltpu
from jax.experimental.pallas import tpu_sc as plsc

_NW = 32


def _make_gather(B0, T, K, D, unroll=32):
    FS = 8
    NSLAB = D // FS
    NQ = _NW // NSLAB
    TOK = T // NQ
    IMG_BLK = 8
    NBLK = B0 // IMG_BLK
    NGRP = IMG_BLK * (TOK // 16)
    mesh = plsc.VectorSubcoreMesh(core_axis_name="c", subcore_axis_name="s")

    @functools.partial(
        pl.kernel,
        mesh=mesh,
        compiler_params=pltpu.CompilerParams(needs_layout_passes=False),
        out_type=jax.ShapeDtypeStruct((B0, D, T), jnp.float32),
        scratch_types=[
            pltpu.VMEM((FS, K), jnp.float32),
            pltpu.VMEM((IMG_BLK, TOK), jnp.int32),
            pltpu.VMEM((IMG_BLK, TOK), jnp.int32),
            pltpu.VMEM((IMG_BLK, FS, TOK), jnp.float32),
            pltpu.VMEM((IMG_BLK, FS, TOK), jnp.float32),
            [pltpu.SemaphoreType.DMA] * 2,
            [pltpu.SemaphoreType.DMA] * 2,
        ],
    )
    def gather_kernel(
        idx_hbm, ct_hbm, out_hbm, trows, idx0, idx1, ob0, ob1, isems, osems
    ):
        wid = lax.axis_index("s") * 2 + lax.axis_index("c")
        slab = wid // NQ
        f0 = slab * FS
        t0 = (wid % NQ) * TOK
        pltpu.sync_copy(ct_hbm.at[pl.ds(f0, FS)], trows)

        idx_bufs = (idx0, idx1)
        obufs = (ob0, ob1)

        def idx_src(img0):
            return idx_hbm.at[pl.ds(img0, IMG_BLK), pl.ds(t0, TOK)]

        def out_dst(img0):
            return out_hbm.at[
                pl.ds(img0, IMG_BLK), pl.ds(f0, FS), pl.ds(t0, TOK)
            ]

        def compute(idx_v, obuf):
            @plsc.parallel_loop(0, NGRP, unroll=unroll)
            def _grp(i, _idx_v=idx_v, _obuf=obuf):
                il = i >> 4
                g = i & 15
                iv = _idx_v[il, pl.ds(g * 16, 16)]
                zero = jnp.zeros_like(iv)
                for fl in range(FS):
                    _obuf[il, fl, pl.ds(g * 16, 16)] = plsc.load_gather(
                        trows, [zero + fl, iv]
                    )

        def step(blk, b, wait_out):
            nxt = jnp.minimum((blk + 1) * IMG_BLK, B0 - IMG_BLK)
            pltpu.async_copy(idx_src(nxt), idx_bufs[1 - b], isems[1 - b])
            pltpu.make_async_copy(idx_src(0), idx_bufs[b], isems[b]).wait()
            if wait_out:
                pltpu.make_async_copy(obufs[b], out_dst(0), osems[b]).wait()
            compute(idx_bufs[b], obufs[b])
            pltpu.async_copy(obufs[b], out_dst(blk * IMG_BLK), osems[b])

        pltpu.async_copy(idx_src(0), idx_bufs[0], isems[0])
        step(0, 0, False)
        step(1, 1, False)

        @pl.loop(1, NBLK // 2)
        def _pair(p):
            blk = p * 2
            step(blk, 0, True)
            step(blk + 1, 1, True)

        pltpu.make_async_copy(idx_src(0), idx_bufs[0], isems[0]).wait()
        pltpu.make_async_copy(obufs[0], out_dst(0), osems[0]).wait()
        pltpu.make_async_copy(obufs[1], out_dst(0), osems[1]).wait()

    return gather_kernel


def kernel(indices, codebook):
    B0, T = indices.shape
    K, D = codebook.shape
    ct = codebook.T
    out = _make_gather(B0, T, K, D)(indices, ct)
    return out.transpose(0, 2, 1)
